# baseline (speedup 1.0000x reference)
import jax
import jax.numpy as jnp
from jax.experimental import pallas as pl
from jax.experimental.pallas import tpu as pltpu

_CHUNK_ROWS = (1, 1, 2, 4, 8, 8, 8, 9, 9)


def _body(col_ref, row_ref, o_ref, colrep_ref, rowrep_ref, asm_ref, sems):
    height, width, _, batch, emb = asm_ref.shape

    colrep_ref[...] = jnp.broadcast_to(
        col_ref[...][:, None, :], (width, batch, emb)
    )
    rowrep_ref[...] = jnp.broadcast_to(
        row_ref[...][:, None, :], (height, batch, emb)
    )

    copies = []
    row0 = 0
    for i, rows in enumerate(_CHUNK_ROWS):
        for h in range(row0, row0 + rows):
            asm_ref[h, :, 0] = colrep_ref[...]
            asm_ref[h, :, 1] = jnp.broadcast_to(
                rowrep_ref[h][None], (width, batch, emb)
            )
        copies.append(pltpu.async_copy(
            asm_ref.at[pl.ds(row0, rows)],
            o_ref.at[pl.ds(row0, rows)],
            sems.at[i],
        ))
        row0 += rows
    for c in copies:
        c.wait()


def kernel(pixel_values, row_embeddings, column_embeddings):
    batch, d_model, height, width = pixel_values.shape
    emb = row_embeddings.shape[1]
    assert sum(_CHUNK_ROWS) == height

    out = pl.pallas_call(
        _body,
        out_specs=pl.BlockSpec(memory_space=pl.ANY),
        out_shape=jax.ShapeDtypeStruct(
            (height, width, 2, batch, emb), jnp.float32
        ),
        scratch_shapes=[
            pltpu.VMEM((width, batch, emb), jnp.float32),
            pltpu.VMEM((height, batch, emb), jnp.float32),
            pltpu.VMEM((height, width, 2, batch, emb), jnp.float32),
            pltpu.SemaphoreType.DMA((len(_CHUNK_ROWS),)),
        ],
    )(column_embeddings, row_embeddings)

    return jnp.transpose(out, (3, 2, 4, 0, 1)).reshape(
        batch, d_model, height, width
    )
